# baseline (speedup 1.0000x reference)
import functools

import jax
import jax.numpy as jnp
from jax import lax
from jax.experimental import pallas as pl
from jax.experimental.pallas import tpu as pltpu
from jax.experimental.pallas import tpu_sc as plsc

_NC = 2
_NS = 16
_L = 16
_LANES = 128
_Q = 4


def _build(B, N, V):
    mesh = plsc.VectorSubcoreMesh(
        core_axis_name="c", subcore_axis_name="s",
        num_cores=_NC, num_subcores=_NS,
    )
    rows_per_q = N // _Q
    q_chunks = rows_per_q // _L
    n_scat = rows_per_q // _LANES
    trash0 = _NS * rows_per_q
    sh_len = trash0 + _L

    @functools.partial(
        pl.kernel,
        out_type=(
            jax.ShapeDtypeStruct((B, N), jnp.float32),
            jax.ShapeDtypeStruct((B, N), jnp.int32),
        ),
        mesh=mesh,
        compiler_params=pltpu.CompilerParams(needs_layout_passes=False),
        scratch_types=(
            pltpu.VMEM((rows_per_q, _LANES), jnp.float32),
            pltpu.VMEM((rows_per_q,), jnp.float32),
            pltpu.VMEM((rows_per_q,), jnp.int32),
            pltpu.VMEM((n_scat, _LANES), jnp.int32),
            pltpu.VMEM((rows_per_q,), jnp.int32),
            pltpu.VMEM((_L,), jnp.int32),
            pltpu.VMEM((_L,), jnp.int32),
            pltpu.VMEM_SHARED((_NS, _L), jnp.int32),
            pltpu.VMEM_SHARED((sh_len,), jnp.int32),
            pltpu.SemaphoreType.DMA,
            pltpu.SemaphoreType.DMA,
        ),
    )
    def fk(oh_ref, idvec_ref, sel_ref, idx_ref,
           blk_v, cvals_v, rvals_v, pvals_v, zq_v, id_v, cnt_v,
           cnt_sh, idx_sh, sem, sem2):
        c = lax.axis_index("c")
        s = lax.axis_index("s")
        b = c * (B // _NC) + s // _Q
        q = s % _Q

        id_cp = pltpu.async_copy(idvec_ref, id_v, sem2)

        def zfill(k, _):
            zq_v[pl.ds(k * _L, _L)] = jnp.zeros((_L,), jnp.int32)
            return 0

        lax.fori_loop(0, q_chunks, zfill, 0)

        id_cp.wait()
        sid = jnp.max(id_v[...])
        sid_base = pl.multiple_of((sid // _LANES) * _LANES, _LANES)
        off = jnp.full((_L,), sid % _LANES, dtype=jnp.int32)
        lane = lax.iota(jnp.int32, _L)
        last = jnp.full((_L,), _L - 1, dtype=jnp.int32)
        row0 = pl.multiple_of(q * rows_per_q, rows_per_q)
        reg0 = pl.multiple_of(s * rows_per_q, rows_per_q)

        half = rows_per_q // 2
        blk_cps = []
        for h in range(2):
            r0 = pl.multiple_of(row0 + h * half, half)
            blk_cps.append(pltpu.async_copy(
                oh_ref.at[b, pl.ds(r0, half), pl.ds(sid_base, _LANES)],
                blk_v.at[pl.ds(h * half, half)], sem))

        pltpu.sync_copy(zq_v, idx_sh.at[pl.ds(reg0, rows_per_q)])

        br0v = jnp.full((_L,), (s - q) * rows_per_q, dtype=jnp.int32)
        trashv = jnp.full((_L,), trash0, dtype=jnp.int32) + lane
        ones = jnp.ones((_L,), jnp.int32)
        zeros = jnp.zeros((_L,), jnp.int32)
        carryv = zeros
        for cch in range(n_scat):
            if cch % (n_scat // 2) == 0:
                blk_cps[cch // (n_scat // 2)].wait()

            def extract(j2, cv):
                j = cch * (_LANES // _L) + j2
                rows = j * _L + lane
                vals = plsc.load_gather(blk_v, [rows, off])
                cvals_v[pl.ds(j * _L, _L)] = vals
                rvals_v[pl.ds(j * _L, _L)] = row0 + rows
                mi = jnp.where(vals != 0.0, ones, zeros)
                cs = plsc.cumsum(mi)
                pos = (cv + cs) - mi
                pvals_v[cch, pl.ds(j2 * _L, _L)] = jnp.where(
                    vals != 0.0, br0v + pos, trashv)
                cs_last = lax.gather(
                    cs, last[:, None],
                    dimension_numbers=lax.GatherDimensionNumbers(
                        offset_dims=(), collapsed_slice_dims=(0,),
                        start_index_map=(0,)),
                    slice_sizes=(1,),
                    mode=lax.GatherScatterMode.PROMISE_IN_BOUNDS)
                return cv + cs_last

            carryv = lax.fori_loop(0, _LANES // _L, extract, carryv)

        cnt_v[...] = carryv
        pltpu.sync_copy(cnt_v, cnt_sh.at[s])
        sel_cp = pltpu.async_copy(
            cvals_v, sel_ref.at[b, pl.ds(row0, rows_per_q)], sem2)

        plsc.subcore_barrier()

        qv = jnp.full((_L,), q, dtype=jnp.int32)
        basev = zeros
        for i in range(_Q - 1):
            pltpu.sync_copy(cnt_sh.at[s - q + i], cnt_v)
            iv = jnp.full((_L,), i, dtype=jnp.int32)
            basev = basev + jnp.where(iv < qv, cnt_v[...], zeros)

        tthr = jnp.full((_L,), trash0, dtype=jnp.int32)
        for cch in range(n_scat):
            def rebase(j2, _):
                p = pvals_v[cch, pl.ds(j2 * _L, _L)]
                pvals_v[cch, pl.ds(j2 * _L, _L)] = p + jnp.where(
                    p < tthr, basev, zeros)
                return 0

            lax.fori_loop(0, _LANES // _L, rebase, 0)
            pltpu.sync_copy(rvals_v.at[pl.ds(cch * _LANES, _LANES)],
                            idx_sh.at[pvals_v.at[cch]])

        sel_cp.wait()
        plsc.subcore_barrier()

        @pl.when(q == 0)
        def _flush():
            pltpu.sync_copy(idx_sh.at[pl.ds(reg0, N)], idx_ref.at[b])

    return fk


def kernel(one_hot, id):
    B, N, V = one_hot.shape
    id_vec = jnp.full((_L,), id, dtype=jnp.int32)
    sel, idx = _build(B, N, V)(one_hot, id_vec)
    return sel.reshape(B, N, 1, 1), idx


# speedup vs baseline: 1.0010x; 1.0010x over previous
import functools

import jax
import jax.numpy as jnp
from jax import lax
from jax.experimental import pallas as pl
from jax.experimental.pallas import tpu as pltpu
from jax.experimental.pallas import tpu_sc as plsc

_NC = 2
_NS = 16
_L = 16
_LANES = 128
_Q = 4


def _build(B, N, V):
    mesh = plsc.VectorSubcoreMesh(
        core_axis_name="c", subcore_axis_name="s",
        num_cores=_NC, num_subcores=_NS,
    )
    rows_per_q = N // _Q
    q_chunks = rows_per_q // _L
    n_scat = rows_per_q // _LANES
    trash0 = _NS * rows_per_q
    sh_len = trash0 + _L

    @functools.partial(
        pl.kernel,
        out_type=(
            jax.ShapeDtypeStruct((B, N), jnp.float32),
            jax.ShapeDtypeStruct((B, N), jnp.int32),
        ),
        mesh=mesh,
        compiler_params=pltpu.CompilerParams(needs_layout_passes=False),
        scratch_types=(
            pltpu.VMEM((rows_per_q, _LANES), jnp.float32),
            pltpu.VMEM((rows_per_q,), jnp.float32),
            pltpu.VMEM((rows_per_q,), jnp.int32),
            pltpu.VMEM((n_scat, _LANES), jnp.int32),
            pltpu.VMEM((rows_per_q,), jnp.int32),
            pltpu.VMEM((_L,), jnp.int32),
            pltpu.VMEM((_L,), jnp.int32),
            pltpu.VMEM_SHARED((_NS, _L), jnp.int32),
            pltpu.VMEM_SHARED((sh_len,), jnp.int32),
            pltpu.SemaphoreType.DMA,
            pltpu.SemaphoreType.DMA,
        ),
    )
    def fk(oh_ref, idvec_ref, sel_ref, idx_ref,
           blk_v, cvals_v, rvals_v, pvals_v, zq_v, id_v, cnt_v,
           cnt_sh, idx_sh, sem, sem2):
        c = lax.axis_index("c")
        s = lax.axis_index("s")
        b = c * (B // _NC) + s // _Q
        q = s % _Q

        id_cp = pltpu.async_copy(idvec_ref, id_v, sem2)

        def zfill(k, _):
            zq_v[pl.ds(k * _L, _L)] = jnp.zeros((_L,), jnp.int32)
            return 0

        lax.fori_loop(0, q_chunks, zfill, 0)

        id_cp.wait()
        sid = jnp.max(id_v[...])
        sid_base = pl.multiple_of((sid // _LANES) * _LANES, _LANES)
        off = jnp.full((_L,), sid % _LANES, dtype=jnp.int32)
        lane = lax.iota(jnp.int32, _L)
        last = jnp.full((_L,), _L - 1, dtype=jnp.int32)
        row0 = pl.multiple_of(q * rows_per_q, rows_per_q)
        reg0 = pl.multiple_of(s * rows_per_q, rows_per_q)

        half = rows_per_q // 2
        blk_cps = []
        for h in range(2):
            r0 = pl.multiple_of(row0 + h * half, half)
            blk_cps.append(pltpu.async_copy(
                oh_ref.at[b, pl.ds(r0, half), pl.ds(sid_base, _LANES)],
                blk_v.at[pl.ds(h * half, half)], sem))

        pltpu.sync_copy(zq_v, idx_sh.at[pl.ds(reg0, rows_per_q)])

        br0v = jnp.full((_L,), (s - q) * rows_per_q, dtype=jnp.int32)
        trashv = jnp.full((_L,), trash0, dtype=jnp.int32) + lane
        ones = jnp.ones((_L,), jnp.int32)
        zeros = jnp.zeros((_L,), jnp.int32)
        carryv = zeros
        for cch in range(n_scat):
            if cch % (n_scat // 2) == 0:
                blk_cps[cch // (n_scat // 2)].wait()

            def extract(j2, cv):
                j = cch * (_LANES // _L) + j2
                rows = j * _L + lane
                vals = plsc.load_gather(blk_v, [rows, off])
                cvals_v[pl.ds(j * _L, _L)] = vals
                rvals_v[pl.ds(j * _L, _L)] = row0 + rows
                mi = jnp.where(vals != 0.0, ones, zeros)
                cs = plsc.cumsum(mi)
                pos = (cv + cs) - mi
                pvals_v[cch, pl.ds(j2 * _L, _L)] = jnp.where(
                    vals != 0.0, br0v + pos, trashv)
                cs_last = lax.gather(
                    cs, last[:, None],
                    dimension_numbers=lax.GatherDimensionNumbers(
                        offset_dims=(), collapsed_slice_dims=(0,),
                        start_index_map=(0,)),
                    slice_sizes=(1,),
                    mode=lax.GatherScatterMode.PROMISE_IN_BOUNDS)
                return cv + cs_last

            carryv = lax.fori_loop(0, _LANES // _L, extract, carryv)

        cnt_v[...] = carryv
        pltpu.sync_copy(cnt_v, cnt_sh.at[s])
        sel_cp = pltpu.async_copy(
            cvals_v, sel_ref.at[b, pl.ds(row0, rows_per_q)], sem2)

        plsc.subcore_barrier()

        qv = jnp.full((_L,), q, dtype=jnp.int32)
        basev = zeros
        for i in range(_Q - 1):
            pltpu.sync_copy(cnt_sh.at[s - q + i], cnt_v)
            iv = jnp.full((_L,), i, dtype=jnp.int32)
            basev = basev + jnp.where(iv < qv, cnt_v[...], zeros)

        tthr = jnp.full((_L,), trash0, dtype=jnp.int32)
        for cch in range(n_scat):
            def rebase(j2, _):
                p = pvals_v[cch, pl.ds(j2 * _L, _L)]
                pvals_v[cch, pl.ds(j2 * _L, _L)] = p + jnp.where(
                    p < tthr, basev, zeros)
                return 0

            lax.fori_loop(0, _LANES // _L, rebase, 0)
            pltpu.sync_copy(rvals_v.at[pl.ds(cch * _LANES, _LANES)],
                            idx_sh.at[pvals_v.at[cch]])

        sel_cp.wait()
        plsc.subcore_barrier()

        breg = pl.multiple_of((s - q) * rows_per_q + q * rows_per_q, rows_per_q)
        pltpu.sync_copy(idx_sh.at[pl.ds(breg, rows_per_q)],
                        idx_ref.at[b, pl.ds(row0, rows_per_q)])

    return fk


def kernel(one_hot, id):
    B, N, V = one_hot.shape
    id_vec = jnp.full((_L,), id, dtype=jnp.int32)
    sel, idx = _build(B, N, V)(one_hot, id_vec)
    return sel.reshape(B, N, 1, 1), idx


# speedup vs baseline: 1.0045x; 1.0035x over previous
import functools

import jax
import jax.numpy as jnp
from jax import lax
from jax.experimental import pallas as pl
from jax.experimental.pallas import tpu as pltpu
from jax.experimental.pallas import tpu_sc as plsc

_NC = 2
_NS = 16
_L = 16
_LANES = 128
_Q = 4


def _build_tc_extract(B, N, V):
    def tk(sid_ref, x_ref, col_ref):
        off = sid_ref[0] % _LANES
        x = x_ref[...]
        lanes = lax.broadcasted_iota(jnp.int32, (1, N, _LANES), 2)
        col_ref[...] = jnp.sum(
            jnp.where(lanes == off, x, 0.0), axis=2, keepdims=True
        ).transpose(0, 2, 1)

    return pl.pallas_call(
        tk,
        grid_spec=pltpu.PrefetchScalarGridSpec(
            num_scalar_prefetch=1,
            grid=(B,),
            in_specs=[pl.BlockSpec((1, N, _LANES),
                                   lambda i, sid: (i, 0, sid[0] // _LANES))],
            out_specs=pl.BlockSpec((1, 1, N), lambda i, sid: (i, 0, 0)),
        ),
        out_shape=jax.ShapeDtypeStruct((B, 1, N), jnp.float32),
    )


def _build_sc_compact(B, N):
    mesh = plsc.VectorSubcoreMesh(
        core_axis_name="c", subcore_axis_name="s",
        num_cores=_NC, num_subcores=_NS,
    )
    rows_per_q = N // _Q
    q_chunks = rows_per_q // _L
    n_scat = rows_per_q // _LANES
    trash0 = _NS * rows_per_q
    sh_len = trash0 + _L

    @functools.partial(
        pl.kernel,
        out_type=jax.ShapeDtypeStruct((B, N), jnp.int32),
        mesh=mesh,
        compiler_params=pltpu.CompilerParams(needs_layout_passes=False),
        scratch_types=(
            pltpu.VMEM((rows_per_q,), jnp.float32),
            pltpu.VMEM((rows_per_q,), jnp.int32),
            pltpu.VMEM((n_scat, _LANES), jnp.int32),
            pltpu.VMEM((rows_per_q,), jnp.int32),
            pltpu.VMEM((_L,), jnp.int32),
            pltpu.VMEM_SHARED((_NS, _L), jnp.int32),
            pltpu.VMEM_SHARED((sh_len,), jnp.int32),
            pltpu.SemaphoreType.DMA,
        ),
    )
    def fk(col_ref, idx_ref, cvals_v, rvals_v, pvals_v, zq_v, cnt_v,
           cnt_sh, idx_sh, sem):
        c = lax.axis_index("c")
        s = lax.axis_index("s")
        b = c * (B // _NC) + s // _Q
        q = s % _Q

        lane = lax.iota(jnp.int32, _L)
        last = jnp.full((_L,), _L - 1, dtype=jnp.int32)
        row0 = pl.multiple_of(q * rows_per_q, rows_per_q)
        reg0 = pl.multiple_of(s * rows_per_q, rows_per_q)

        col_cp = pltpu.async_copy(
            col_ref.at[b, 0, pl.ds(row0, rows_per_q)], cvals_v, sem)

        def zfill(k, _):
            zq_v[pl.ds(k * _L, _L)] = jnp.zeros((_L,), jnp.int32)
            return 0

        lax.fori_loop(0, q_chunks, zfill, 0)
        pltpu.sync_copy(zq_v, idx_sh.at[pl.ds(reg0, rows_per_q)])

        col_cp.wait()

        br0v = jnp.full((_L,), (s - q) * rows_per_q, dtype=jnp.int32)
        trashv = jnp.full((_L,), trash0, dtype=jnp.int32) + lane
        ones = jnp.ones((_L,), jnp.int32)
        zeros = jnp.zeros((_L,), jnp.int32)
        carryv = zeros
        for cch in range(n_scat):
            def scan(j2, cv):
                j = cch * (_LANES // _L) + j2
                rows = j * _L + lane
                vals = cvals_v[pl.ds(j * _L, _L)]
                rvals_v[pl.ds(j * _L, _L)] = row0 + rows
                mi = jnp.where(vals != 0.0, ones, zeros)
                cs = plsc.cumsum(mi)
                pos = (cv + cs) - mi
                pvals_v[cch, pl.ds(j2 * _L, _L)] = jnp.where(
                    vals != 0.0, br0v + pos, trashv)
                cs_last = lax.gather(
                    cs, last[:, None],
                    dimension_numbers=lax.GatherDimensionNumbers(
                        offset_dims=(), collapsed_slice_dims=(0,),
                        start_index_map=(0,)),
                    slice_sizes=(1,),
                    mode=lax.GatherScatterMode.PROMISE_IN_BOUNDS)
                return cv + cs_last

            carryv = lax.fori_loop(0, _LANES // _L, scan, carryv)

        cnt_v[...] = carryv
        pltpu.sync_copy(cnt_v, cnt_sh.at[s])

        plsc.subcore_barrier()

        qv = jnp.full((_L,), q, dtype=jnp.int32)
        basev = zeros
        for i in range(_Q - 1):
            pltpu.sync_copy(cnt_sh.at[s - q + i], cnt_v)
            iv = jnp.full((_L,), i, dtype=jnp.int32)
            basev = basev + jnp.where(iv < qv, cnt_v[...], zeros)

        tthr = jnp.full((_L,), trash0, dtype=jnp.int32)
        for cch in range(n_scat):
            def rebase(j2, _):
                p = pvals_v[cch, pl.ds(j2 * _L, _L)]
                pvals_v[cch, pl.ds(j2 * _L, _L)] = p + jnp.where(
                    p < tthr, basev, zeros)
                return 0

            lax.fori_loop(0, _LANES // _L, rebase, 0)
            pltpu.sync_copy(rvals_v.at[pl.ds(cch * _LANES, _LANES)],
                            idx_sh.at[pvals_v.at[cch]])

        plsc.subcore_barrier()

        breg = pl.multiple_of((s - q) * rows_per_q + q * rows_per_q, rows_per_q)
        pltpu.sync_copy(idx_sh.at[pl.ds(breg, rows_per_q)],
                        idx_ref.at[b, pl.ds(row0, rows_per_q)])

    return fk


def kernel(one_hot, id):
    B, N, V = one_hot.shape
    sid_arr = jnp.asarray(id, dtype=jnp.int32).reshape(1)
    col = _build_tc_extract(B, N, V)(sid_arr, one_hot)
    idx = _build_sc_compact(B, N)(col)
    return col.reshape(B, N, 1, 1), idx


# speedup vs baseline: 1.0539x; 1.0492x over previous
import functools

import jax
import jax.numpy as jnp
from jax import lax
from jax.experimental import pallas as pl
from jax.experimental.pallas import tpu as pltpu
from jax.experimental.pallas import tpu_sc as plsc

_NC = 2
_NS = 16
_L = 16
_LANES = 128
_Q = 4


def _build_tc_extract(B, N, V):
    def tk(sid_ref, x_ref, col_ref):
        off = sid_ref[0] % _LANES
        x = x_ref[0]
        sel = (lax.broadcasted_iota(jnp.int32, (1, _LANES), 1)
               == off).astype(jnp.float32)
        col_ref[0] = lax.dot_general(
            sel, x, (((1,), (1,)), ((), ())),
            preferred_element_type=jnp.float32)

    return pl.pallas_call(
        tk,
        grid_spec=pltpu.PrefetchScalarGridSpec(
            num_scalar_prefetch=1,
            grid=(B,),
            in_specs=[pl.BlockSpec((1, N, _LANES),
                                   lambda i, sid: (i, 0, sid[0] // _LANES))],
            out_specs=pl.BlockSpec((1, 1, N), lambda i, sid: (i, 0, 0)),
        ),
        out_shape=jax.ShapeDtypeStruct((B, 1, N), jnp.float32),
    )


def _build_sc_compact(B, N):
    mesh = plsc.VectorSubcoreMesh(
        core_axis_name="c", subcore_axis_name="s",
        num_cores=_NC, num_subcores=_NS,
    )
    rows_per_q = N // _Q
    q_chunks = rows_per_q // _L
    n_scat = rows_per_q // _LANES
    trash0 = _NS * rows_per_q
    sh_len = trash0 + _L

    @functools.partial(
        pl.kernel,
        out_type=jax.ShapeDtypeStruct((B, N), jnp.int32),
        mesh=mesh,
        compiler_params=pltpu.CompilerParams(needs_layout_passes=False),
        scratch_types=(
            pltpu.VMEM((rows_per_q,), jnp.float32),
            pltpu.VMEM((rows_per_q,), jnp.int32),
            pltpu.VMEM((n_scat, _LANES), jnp.int32),
            pltpu.VMEM((rows_per_q,), jnp.int32),
            pltpu.VMEM((_L,), jnp.int32),
            pltpu.VMEM_SHARED((_NS, _L), jnp.int32),
            pltpu.VMEM_SHARED((sh_len,), jnp.int32),
            pltpu.SemaphoreType.DMA,
        ),
    )
    def fk(col_ref, idx_ref, cvals_v, rvals_v, pvals_v, zq_v, cnt_v,
           cnt_sh, idx_sh, sem):
        c = lax.axis_index("c")
        s = lax.axis_index("s")
        b = c * (B // _NC) + s // _Q
        q = s % _Q

        lane = lax.iota(jnp.int32, _L)
        last = jnp.full((_L,), _L - 1, dtype=jnp.int32)
        row0 = pl.multiple_of(q * rows_per_q, rows_per_q)
        reg0 = pl.multiple_of(s * rows_per_q, rows_per_q)

        col_cp = pltpu.async_copy(
            col_ref.at[b, 0, pl.ds(row0, rows_per_q)], cvals_v, sem)

        def zfill(k, _):
            zq_v[pl.ds(k * _L, _L)] = jnp.zeros((_L,), jnp.int32)
            return 0

        lax.fori_loop(0, q_chunks, zfill, 0)
        pltpu.sync_copy(zq_v, idx_sh.at[pl.ds(reg0, rows_per_q)])

        col_cp.wait()

        br0v = jnp.full((_L,), (s - q) * rows_per_q, dtype=jnp.int32)
        trashv = jnp.full((_L,), trash0, dtype=jnp.int32) + lane
        ones = jnp.ones((_L,), jnp.int32)
        zeros = jnp.zeros((_L,), jnp.int32)
        carryv = zeros
        for cch in range(n_scat):
            def scan(j2, cv):
                j = cch * (_LANES // _L) + j2
                rows = j * _L + lane
                vals = cvals_v[pl.ds(j * _L, _L)]
                rvals_v[pl.ds(j * _L, _L)] = row0 + rows
                mi = jnp.where(vals != 0.0, ones, zeros)
                cs = plsc.cumsum(mi)
                pos = (cv + cs) - mi
                pvals_v[cch, pl.ds(j2 * _L, _L)] = jnp.where(
                    vals != 0.0, br0v + pos, trashv)
                cs_last = lax.gather(
                    cs, last[:, None],
                    dimension_numbers=lax.GatherDimensionNumbers(
                        offset_dims=(), collapsed_slice_dims=(0,),
                        start_index_map=(0,)),
                    slice_sizes=(1,),
                    mode=lax.GatherScatterMode.PROMISE_IN_BOUNDS)
                return cv + cs_last

            carryv = lax.fori_loop(0, _LANES // _L, scan, carryv)

        cnt_v[...] = carryv
        pltpu.sync_copy(cnt_v, cnt_sh.at[s])

        plsc.subcore_barrier()

        qv = jnp.full((_L,), q, dtype=jnp.int32)
        basev = zeros
        for i in range(_Q - 1):
            pltpu.sync_copy(cnt_sh.at[s - q + i], cnt_v)
            iv = jnp.full((_L,), i, dtype=jnp.int32)
            basev = basev + jnp.where(iv < qv, cnt_v[...], zeros)

        tthr = jnp.full((_L,), trash0, dtype=jnp.int32)
        for cch in range(n_scat):
            def rebase(j2, _):
                p = pvals_v[cch, pl.ds(j2 * _L, _L)]
                pvals_v[cch, pl.ds(j2 * _L, _L)] = p + jnp.where(
                    p < tthr, basev, zeros)
                return 0

            lax.fori_loop(0, _LANES // _L, rebase, 0)
            pltpu.sync_copy(rvals_v.at[pl.ds(cch * _LANES, _LANES)],
                            idx_sh.at[pvals_v.at[cch]])

        plsc.subcore_barrier()

        breg = pl.multiple_of((s - q) * rows_per_q + q * rows_per_q, rows_per_q)
        pltpu.sync_copy(idx_sh.at[pl.ds(breg, rows_per_q)],
                        idx_ref.at[b, pl.ds(row0, rows_per_q)])

    return fk


def kernel(one_hot, id):
    B, N, V = one_hot.shape
    sid_arr = jnp.asarray(id, dtype=jnp.int32).reshape(1)
    col = _build_tc_extract(B, N, V)(sid_arr, one_hot)
    idx = _build_sc_compact(B, N)(col)
    return col.reshape(B, N, 1, 1), idx
